# Initial kernel scaffold; baseline (speedup 1.0000x reference)
import jax
import jax.numpy as jnp
from jax.experimental import pallas as pl


def kernel(x, edge_index, edge_type, weight, root, bias):
    raise NotImplementedError("write your pallas kernel here")


# speedup vs baseline: 4.9533x; 4.9533x over previous
import functools

import jax
import jax.numpy as jnp
from jax import lax
from jax.experimental import pallas as pl
from jax.experimental.pallas import tpu as pltpu
from jax.experimental.pallas import tpu_sc as plsc

N = 10000
NP = 10240
E = 160000
EP = 163840
D = 256
R = 8

NC = 2
NS = 16
L = 16

KEYS = R * NP
EPW = EP // (NC * NS)
NB = EPW // 128
NCHUNK = D // L
CPC = NCHUNK // NC
SLICE = KEYS // NS


NNB = EP // 128
NBT = NNB // NS
BSUB = 10
NSUB = NBT // BSUB
ESUB = BSUB * 128


def _sc_scatter(xcf, src2, dst2, et2):
  mesh = plsc.VectorSubcoreMesh(core_axis_name="c", subcore_axis_name="s")

  @functools.partial(
      pl.kernel,
      out_type=(
          jax.ShapeDtypeStruct((NCHUNK, KEYS, L), jnp.float32),
          jax.ShapeDtypeStruct((KEYS,), jnp.float32),
      ),
      mesh=mesh,
      scratch_types=(
          pltpu.VMEM_SHARED((KEYS + 8, L), jnp.float32),
          pltpu.VMEM_SHARED((KEYS + 8,), jnp.float32),
          pltpu.VMEM((ESUB, L), jnp.float32),
          pltpu.VMEM((NBT, 128), jnp.int32),
          pltpu.VMEM((NBT, 128), jnp.int32),
          pltpu.VMEM((128,), jnp.float32),
          pltpu.VMEM((SLICE // 16,), jnp.float32),
          pltpu.VMEM((SLICE // 32, L), jnp.float32),
          pltpu.SemaphoreType.DMA,
      ),
      compiler_params=pltpu.CompilerParams(use_tc_tiling_on_sc=False),
  )
  def body(xcf_h, src_h, dst_h, et_h, a_h, c_h,
           acc, cacc, buf, gidx2, key2, ones, z1, z2, sem):
    core = lax.axis_index("c")
    sid = lax.axis_index("s")
    jbase = sid * NBT
    row0 = sid * SLICE

    def fill(i, _):
      ones[pl.ds(i * L, L)] = jnp.full((L,), 1.0, jnp.float32)
      return 0
    lax.fori_loop(0, 128 // L, fill, 0)

    def fillz1(i, _):
      z1[pl.ds(i * L, L)] = jnp.full((L,), 0.0, jnp.float32)
      return 0
    lax.fori_loop(0, SLICE // 16 // L, fillz1, 0)

    def fillz2(i, _):
      z2[i, :] = jnp.full((L,), 0.0, jnp.float32)
      return 0
    lax.fori_loop(0, SLICE // 32, fillz2, 0)

    pltpu.sync_copy(et_h.at[pl.ds(jbase, NBT)], gidx2)
    pltpu.sync_copy(dst_h.at[pl.ds(jbase, NBT)], key2)

    def mkkey(i, _):
      j = i // (128 // L)
      t = i % (128 // L)
      sl = pl.ds(t * L, L)
      key2[j, sl] = gidx2[j, sl] * NP + key2[j, sl]
      return 0
    lax.fori_loop(0, NBT * (128 // L), mkkey, 0)

    @pl.when(core == 0)
    def _counts():
      def zcnt(t, _):
        pltpu.sync_copy(z1, cacc.at[pl.ds(row0 + t * (SLICE // 16),
                                          SLICE // 16)])
        return 0
      lax.fori_loop(0, 16, zcnt, 0)
      plsc.subcore_barrier()

      def cnt_scatter(j, _):
        pltpu.sync_copy(ones, cacc.at[key2.at[j]], add=True)
        return 0
      lax.fori_loop(0, NBT, cnt_scatter, 0)
      plsc.subcore_barrier()
      pltpu.sync_copy(cacc.at[pl.ds(row0, SLICE)],
                      c_h.at[pl.ds(row0, SLICE)])

    for p in range(CPC):
      c = core * CPC + p

      pltpu.sync_copy(src_h.at[pl.ds(jbase, NBT)], gidx2)

      def mkgidx(i, _):
        j = i // (128 // L)
        t = i % (128 // L)
        sl = pl.ds(t * L, L)
        gidx2[j, sl] = gidx2[j, sl] + c * N
        return 0
      lax.fori_loop(0, NBT * (128 // L), mkgidx, 0)

      def zslice(t, _):
        pltpu.sync_copy(z2, acc.at[pl.ds(row0 + t * (SLICE // 32),
                                         SLICE // 32)])
        return 0
      lax.fori_loop(0, 32, zslice, 0)
      plsc.subcore_barrier()

      for h in range(NSUB):
        def fire_gather(j, _):
          pltpu.async_copy(xcf_h.at[gidx2.at[h * BSUB + j]],
                           buf.at[pl.ds(j * 128, 128)], sem)
          return 0
        lax.fori_loop(0, BSUB, fire_gather, 0)
        pltpu.make_async_copy(xcf_h.at[pl.ds(0, ESUB)], buf, sem).wait()

        def scatter(j, _):
          pltpu.sync_copy(buf.at[pl.ds(j * 128, 128)],
                          acc.at[key2.at[h * BSUB + j]], add=True)
          return 0
        lax.fori_loop(0, BSUB, scatter, 0)

      plsc.subcore_barrier()
      pltpu.sync_copy(acc.at[pl.ds(row0, SLICE)],
                      a_h.at[c, pl.ds(row0, SLICE)])
      plsc.subcore_barrier()

  return body(xcf, src2, dst2, et2)


def _tc_dense(xp, a3, cc, weight, root, bias2):
  BN = 256
  grid = (NP // BN,)

  def body(x_ref, a_ref, c_ref, w_ref, root_ref, b_ref, o_ref):
    cnt = c_ref[...]
    inv = 1.0 / jnp.maximum(cnt, 1.0)
    acc = jnp.dot(x_ref[...], root_ref[...],
                  preferred_element_type=jnp.float32,
                  precision=lax.Precision.HIGHEST)
    for r in range(R):
      acc = acc + jnp.dot(a_ref[r] * inv[r][:, None], w_ref[r],
                          preferred_element_type=jnp.float32,
                          precision=lax.Precision.HIGHEST)
    o_ref[...] = acc + b_ref[...]

  return pl.pallas_call(
      body,
      grid=grid,
      in_specs=[
          pl.BlockSpec((BN, D), lambda i: (i, 0)),
          pl.BlockSpec((R, BN, D), lambda i: (0, i, 0)),
          pl.BlockSpec((R, BN), lambda i: (0, i)),
          pl.BlockSpec((R, D, D), lambda i: (0, 0, 0)),
          pl.BlockSpec((D, D), lambda i: (0, 0)),
          pl.BlockSpec((1, D), lambda i: (0, 0)),
      ],
      out_specs=pl.BlockSpec((BN, D), lambda i: (i, 0)),
      out_shape=jax.ShapeDtypeStruct((NP, D), jnp.float32),
  )(xp, a3, cc, weight, root, bias2)


def kernel(x, edge_index, edge_type, weight, root, bias):
  xcf = x.reshape(N, NCHUNK, L).swapaxes(0, 1).reshape(NCHUNK * N, L)
  src = jnp.pad(edge_index[0].astype(jnp.int32),
                (0, EP - E)).reshape(EP // 128, 128)
  dst = jnp.pad(edge_index[1].astype(jnp.int32),
                (0, EP - E)).reshape(EP // 128, 128)
  et = jnp.pad(edge_type.astype(jnp.int32), (0, EP - E),
               constant_values=R).reshape(EP // 128, 128)

  a, c = _sc_scatter(xcf, src, dst, et)

  a3 = a.transpose(1, 0, 2).reshape(R, NP, D)
  cc = c.reshape(R, NP)
  xp = jnp.pad(x, ((0, NP - N), (0, 0)))
  out = _tc_dense(xp, a3, cc, weight, root, bias.reshape(1, D))
  return out[:N]
